# baseline (speedup 1.0000x reference)
import functools

import jax
import jax.numpy as jnp
from jax.experimental import pallas as pl

NUM_QUANT = 8
CODEBOOK_LEN = 8192
LATENT_DIM = 32
TOK_TILE = 512


def _rvq_kernel(x_ref, cbt16_ref, cbhl_ref, c2_ref,
                zq_ref, loss_ref):
    r = x_ref[...]
    t = r.shape[0]
    half = CODEBOOK_LEN // 2
    iota = jax.lax.broadcasted_iota(jnp.int32, (t, CODEBOOK_LEN), 1)
    iota_h = jax.lax.broadcasted_iota(jnp.int32, (t, half), 1)
    zq_acc = jnp.zeros_like(r)
    loss_acc = jnp.float32(0.0)
    for s in range(NUM_QUANT):
        a = jnp.sum(r * r, axis=1, keepdims=True)
        b = jax.lax.dot_general(
            r.astype(jnp.bfloat16), cbt16_ref[s],
            (((1,), (0,)), ((), ())), preferred_element_type=jnp.float32)
        t1 = a - 2.0 * b
        sq = t1 + c2_ref[s][None, :]
        sq0 = sq[:, :half]
        sq1 = sq[:, half:]
        m0 = jnp.min(sq0, axis=1, keepdims=True)
        m1 = jnp.min(sq1, axis=1, keepdims=True)
        i0 = jnp.min(jnp.where(sq0 == m0, iota_h, half), axis=1)
        i1 = jnp.min(jnp.where(sq1 == m1, iota_h, half), axis=1) + half
        m0r = m0[:, 0].astype(jnp.bfloat16).astype(jnp.float32)
        idx = jnp.where(m1[:, 0] <= m0r, i1, i0)
        oh = (iota == idx[:, None]).astype(jnp.bfloat16)
        dn = (((1,), (0,)), ((), ()))
        zq2 = jax.lax.dot_general(oh, cbhl_ref[s], dn,
                                  preferred_element_type=jnp.float32)
        z_q = zq2[:, :LATENT_DIM] + zq2[:, LATENT_DIM:]
        r = r - z_q
        zq_acc = zq_acc + z_q
        loss_acc = loss_acc + jnp.sum(r * r)
    zq_ref[...] = zq_acc

    @pl.when(pl.program_id(0) == 0)
    def _():
        loss_ref[...] = jnp.zeros((1, 1), jnp.float32)

    loss_ref[...] += loss_acc.reshape(1, 1)


@functools.partial(jax.jit, donate_argnums=())
def kernel(x, codebooks):
    b, w, c = x.shape
    n = b * w
    xf = x.reshape(n, c)
    cbt16 = jnp.transpose(codebooks, (0, 2, 1)).astype(jnp.bfloat16)
    cbhi = codebooks.astype(jnp.bfloat16)
    cblo = (codebooks - cbhi.astype(jnp.float32)).astype(jnp.bfloat16)
    cbhl = jnp.concatenate([cbhi, cblo], axis=2)
    c2 = jnp.sum(codebooks * codebooks, axis=2)

    zq, loss = pl.pallas_call(
        _rvq_kernel,
        grid=(n // TOK_TILE,),
        in_specs=[
            pl.BlockSpec((TOK_TILE, c), lambda i: (i, 0)),
            pl.BlockSpec((NUM_QUANT, c, CODEBOOK_LEN), lambda i: (0, 0, 0)),
            pl.BlockSpec((NUM_QUANT, CODEBOOK_LEN, 2 * c), lambda i: (0, 0, 0)),
            pl.BlockSpec((NUM_QUANT, CODEBOOK_LEN), lambda i: (0, 0)),
        ],
        out_specs=[
            pl.BlockSpec((TOK_TILE, c), lambda i: (i, 0)),
            pl.BlockSpec((1, 1), lambda i: (0, 0)),
        ],
        out_shape=[
            jax.ShapeDtypeStruct((n, c), jnp.float32),
            jax.ShapeDtypeStruct((1, 1), jnp.float32),
        ],
    )(xf, cbt16, cbhl, c2)
    scale = jnp.float32(2.75 / (n * c))
    return (zq.reshape(b, w, c), (loss[0, 0] * scale).astype(jnp.float32))


# speedup vs baseline: 1.5761x; 1.5761x over previous
import functools

import jax
import jax.numpy as jnp
from jax.experimental import pallas as pl

NUM_QUANT = 8
CODEBOOK_LEN = 8192
LATENT_DIM = 32
TOK_TILE = 256


def _rvq_kernel(x_ref, cbt16_ref, cbhl_ref, c2_ref,
                zq_ref, loss_ref):
    r = x_ref[...]
    t = r.shape[0]
    half = CODEBOOK_LEN // 2
    iota = jax.lax.broadcasted_iota(jnp.int32, (t, CODEBOOK_LEN), 1)
    iota_h = jax.lax.broadcasted_iota(jnp.int32, (t, half), 1)
    zq_acc = jnp.zeros_like(r)
    loss_acc = jnp.float32(0.0)
    for s in range(NUM_QUANT):
        a = jnp.sum(r * r, axis=1, keepdims=True)
        b = jax.lax.dot_general(
            r.astype(jnp.bfloat16), cbt16_ref[s],
            (((1,), (0,)), ((), ())), preferred_element_type=jnp.float32)
        t1 = a - 2.0 * b
        sq = t1 + c2_ref[s][None, :]
        sq0 = sq[:, :half]
        sq1 = sq[:, half:]
        m0 = jnp.min(sq0, axis=1, keepdims=True)
        m1 = jnp.min(sq1, axis=1, keepdims=True)
        i0 = jnp.min(jnp.where(sq0 == m0, iota_h, half), axis=1)
        i1 = jnp.min(jnp.where(sq1 == m1, iota_h, half), axis=1) + half
        m0r = m0[:, 0].astype(jnp.bfloat16).astype(jnp.float32)
        idx = jnp.where(m1[:, 0] <= m0r, i1, i0)
        oh = (iota == idx[:, None]).astype(jnp.bfloat16)
        dn = (((1,), (0,)), ((), ()))
        zq2 = jax.lax.dot_general(oh, cbhl_ref[s], dn,
                                  preferred_element_type=jnp.float32)
        z_q = zq2[:, :LATENT_DIM] + zq2[:, LATENT_DIM:]
        r = r - z_q
        zq_acc = zq_acc + z_q
        loss_acc = loss_acc + jnp.sum(r * r)
    zq_ref[...] = zq_acc

    @pl.when(pl.program_id(0) == 0)
    def _():
        loss_ref[...] = jnp.zeros((1, 1), jnp.float32)

    loss_ref[...] += loss_acc.reshape(1, 1)


@functools.partial(jax.jit, donate_argnums=())
def kernel(x, codebooks):
    b, w, c = x.shape
    n = b * w
    xf = x.reshape(n, c)
    cbt16 = jnp.transpose(codebooks, (0, 2, 1)).astype(jnp.bfloat16)
    cbhi = codebooks.astype(jnp.bfloat16)
    cblo = (codebooks - cbhi.astype(jnp.float32)).astype(jnp.bfloat16)
    cbhl = jnp.concatenate([cbhi, cblo], axis=2)
    c2 = jnp.sum(codebooks * codebooks, axis=2)

    zq, loss = pl.pallas_call(
        _rvq_kernel,
        grid=(n // TOK_TILE,),
        in_specs=[
            pl.BlockSpec((TOK_TILE, c), lambda i: (i, 0)),
            pl.BlockSpec((NUM_QUANT, c, CODEBOOK_LEN), lambda i: (0, 0, 0)),
            pl.BlockSpec((NUM_QUANT, CODEBOOK_LEN, 2 * c), lambda i: (0, 0, 0)),
            pl.BlockSpec((NUM_QUANT, CODEBOOK_LEN), lambda i: (0, 0)),
        ],
        out_specs=[
            pl.BlockSpec((TOK_TILE, c), lambda i: (i, 0)),
            pl.BlockSpec((1, 1), lambda i: (0, 0)),
        ],
        out_shape=[
            jax.ShapeDtypeStruct((n, c), jnp.float32),
            jax.ShapeDtypeStruct((1, 1), jnp.float32),
        ],
    )(xf, cbt16, cbhl, c2)
    scale = jnp.float32(2.75 / (n * c))
    return (zq.reshape(b, w, c), (loss[0, 0] * scale).astype(jnp.float32))
